# Initial kernel scaffold; baseline (speedup 1.0000x reference)
import functools

import jax
import jax.numpy as jnp
from jax import lax
from jax.experimental import pallas as pl
from jax.experimental.pallas import tpu as pltpu
from jax.experimental.pallas import tpu_sc as plsc

N = 10000
E = 320000
F = 128
L = 4
EPS = 1e-5

NC = 2
NS = 16
NW = NC * NS
C = 80
IDXROWS = E // C
RW = IDXROWS // NW
ROWS_T = N // NS
ZCH = 125
NZ = ROWS_T // ZCH

_mesh = plsc.VectorSubcoreMesh(core_axis_name="c", subcore_axis_name="s")


@functools.partial(
    pl.kernel,
    out_type=jax.ShapeDtypeStruct((NC, N, F), jnp.float32),
    mesh=_mesh,
    scratch_types=[
        pltpu.VMEM_SHARED((N, F), jnp.float32),
        pltpu.VMEM((RW, C), jnp.int32),
        pltpu.VMEM((RW, C), jnp.int32),
        pltpu.VMEM((C, F), jnp.float32),
        pltpu.VMEM((ZCH, F), jnp.float32),
        pltpu.SemaphoreType.DMA,
    ],
)
def _agg_call(u_hbm, src_hbm, dst_hbm, out_hbm, acc, idxs, idxd, rows, zbuf, sem):
    c = lax.axis_index("c")
    s = lax.axis_index("s")
    wid = c * NS + s
    zero16 = jnp.zeros((16,), jnp.float32)

    def _zrow(r, _):
        for j in range(F // 16):
            zbuf[r, pl.ds(j * 16, 16)] = zero16
        return 0

    lax.fori_loop(0, ZCH, _zrow, 0)
    row0 = s * ROWS_T
    for k in range(NZ):
        pltpu.sync_copy(zbuf, acc.at[pl.ds(row0 + k * ZCH, ZCH)])
    plsc.subcore_barrier()

    pltpu.sync_copy(src_hbm.at[pl.ds(wid * RW, RW)], idxs)
    pltpu.sync_copy(dst_hbm.at[pl.ds(wid * RW, RW)], idxd)

    def _edge_chunk(j, _):
        pltpu.async_copy(u_hbm.at[idxs.at[j]], rows, sem).wait()
        pltpu.sync_copy(rows, acc.at[idxd.at[j]], add=True)
        return 0

    lax.fori_loop(0, RW, _edge_chunk, 0)
    plsc.subcore_barrier()

    for k in range(NZ):
        r0 = row0 + k * ZCH
        pltpu.sync_copy(acc.at[pl.ds(r0, ZCH)], zbuf)
        pltpu.sync_copy(zbuf, out_hbm.at[c, pl.ds(r0, ZCH)])


@functools.partial(
    pl.kernel,
    out_type=jax.ShapeDtypeStruct((NC, N, 16), jnp.float32),
    mesh=_mesh,
    scratch_types=[
        pltpu.VMEM_SHARED((N, 16), jnp.float32),
        pltpu.VMEM((RW, C), jnp.int32),
        pltpu.VMEM((C, 16), jnp.float32),
        pltpu.VMEM((ROWS_T, 16), jnp.float32),
        pltpu.SemaphoreType.DMA,
    ],
)
def _deg_call(dst_hbm, out_hbm, acc, idxd, ones, zbuf, sem):
    c = lax.axis_index("c")
    s = lax.axis_index("s")
    wid = c * NS + s
    zero16 = jnp.zeros((16,), jnp.float32)
    one16 = jnp.ones((16,), jnp.float32)

    def _zrow(r, _):
        zbuf[r, pl.ds(0, 16)] = zero16
        return 0

    lax.fori_loop(0, ROWS_T, _zrow, 0)

    def _orow(r, _):
        ones[r, pl.ds(0, 16)] = one16
        return 0

    lax.fori_loop(0, C, _orow, 0)

    row0 = s * ROWS_T
    pltpu.sync_copy(zbuf, acc.at[pl.ds(row0, ROWS_T)])
    plsc.subcore_barrier()

    pltpu.sync_copy(dst_hbm.at[pl.ds(wid * RW, RW)], idxd)

    def _edge_chunk(j, _):
        pltpu.sync_copy(ones, acc.at[idxd.at[j]], add=True)
        return 0

    lax.fori_loop(0, RW, _edge_chunk, 0)
    plsc.subcore_barrier()

    pltpu.sync_copy(acc.at[pl.ds(row0, ROWS_T)], zbuf)
    pltpu.sync_copy(zbuf, out_hbm.at[c, pl.ds(row0, ROWS_T)])


def _bn_relu(h, gamma, beta):
    mean = jnp.sum(h, axis=0, keepdims=True) * (1.0 / N)
    d = h - mean
    var = jnp.sum(d * d, axis=0, keepdims=True) * (1.0 / N)
    t = gamma * d * lax.rsqrt(var + EPS) + beta
    return jnp.maximum(t, 0.0)


def _k0_body(x_ref, We_ref, be_ref, degp_ref, g_ref, bt_ref, W0_ref,
             h_ref, u_ref, dinv_ref):
    h = jnp.dot(x_ref[...], We_ref[...], preferred_element_type=jnp.float32)
    h = h + be_ref[...]
    dsum = degp_ref[0] + degp_ref[1]
    deg = dsum[:, 0:1] + 1.0
    dinv = lax.rsqrt(jnp.maximum(deg, 1e-12))
    t = _bn_relu(h, g_ref[...], bt_ref[...])
    u = jnp.dot(t, W0_ref[...], preferred_element_type=jnp.float32) * dinv
    h_ref[...] = h
    u_ref[...] = u
    dinv_ref[...] = dinv


def _kpre_body(h_ref, acc_ref, u_ref, dinv_ref, b_ref, g_ref, bt_ref, W_ref,
               hn_ref, un_ref):
    dinv = dinv_ref[...]
    h = h_ref[...] + dinv * (acc_ref[0] + acc_ref[1] + u_ref[...]) + b_ref[...]
    t = _bn_relu(h, g_ref[...], bt_ref[...])
    un_ref[...] = jnp.dot(t, W_ref[...], preferred_element_type=jnp.float32) * dinv
    hn_ref[...] = h


def _kfin_body(h_ref, acc_ref, u_ref, dinv_ref, b_ref, g_ref, bt_ref,
               lw_ref, lb_ref, o_ref):
    dinv = dinv_ref[...]
    h = h_ref[...] + dinv * (acc_ref[0] + acc_ref[1] + u_ref[...]) + b_ref[...]
    t = _bn_relu(h, g_ref[...], bt_ref[...])
    o_ref[...] = jnp.dot(t, lw_ref[...], preferred_element_type=jnp.float32) + lb_ref[...]


def _f32(*s):
    return jax.ShapeDtypeStruct(s, jnp.float32)


_k0_call = pl.pallas_call(
    _k0_body, out_shape=(_f32(N, F), _f32(N, F), _f32(N, 1)))
_kpre_call = pl.pallas_call(
    _kpre_body, out_shape=(_f32(N, F), _f32(N, F)))
_kfin_call = pl.pallas_call(
    _kfin_body, out_shape=_f32(N, F))


def kernel(x, edge_index, W_enc, b_enc, conv_W, conv_b, bn_gamma, bn_beta,
           lin_W, lin_b):
    src2d = edge_index[0].reshape(IDXROWS, C)
    dst2d = edge_index[1].reshape(IDXROWS, C)
    deg_parts = _deg_call(dst2d)
    h, u, dinv = _k0_call(x, W_enc, b_enc.reshape(1, F), deg_parts,
                          bn_gamma[0].reshape(1, F), bn_beta[0].reshape(1, F),
                          conv_W[0])
    for i in range(L):
        accs = _agg_call(u, src2d, dst2d)
        if i < L - 1:
            h, u = _kpre_call(h, accs, u, dinv, conv_b[i].reshape(1, F),
                              bn_gamma[i + 1].reshape(1, F),
                              bn_beta[i + 1].reshape(1, F), conv_W[i + 1])
        else:
            out = _kfin_call(h, accs, u, dinv, conv_b[i].reshape(1, F),
                             bn_gamma[L - 1].reshape(1, F),
                             bn_beta[L - 1].reshape(1, F), lin_W, lin_b)
    return out


# speedup vs baseline: 16.5884x; 16.5884x over previous
import functools

import jax
import jax.numpy as jnp
from jax import lax
from jax.experimental import pallas as pl
from jax.experimental.pallas import tpu as pltpu
from jax.experimental.pallas import tpu_sc as plsc

N = 10000
E = 320000
F = 128
L = 4
EPS = 1e-5

NC = 2
NS = 16
NW = NC * NS
C = 125
IDXROWS = E // C
RW = IDXROWS // NW
NPAD = 10240
ROWS_T = NPAD // NS
ZCH = 64
NZ = ROWS_T // ZCH

_mesh = plsc.VectorSubcoreMesh(core_axis_name="c", subcore_axis_name="s")


@functools.partial(
    pl.kernel,
    out_type=jax.ShapeDtypeStruct((NC, NPAD, F), jnp.float32),
    mesh=_mesh,
    scratch_types=[
        pltpu.VMEM_SHARED((NPAD, F), jnp.float32),
        pltpu.VMEM((RW, C), jnp.int32),
        pltpu.VMEM((RW, C), jnp.int32),
        pltpu.VMEM((C, F), jnp.float32),
        pltpu.VMEM((ZCH, F), jnp.float32),
        pltpu.SemaphoreType.DMA,
    ],
)
def _agg_call(u_hbm, src_hbm, dst_hbm, out_hbm, acc, idxs, idxd, rows, zbuf, sem):
    c = lax.axis_index("c")
    s = lax.axis_index("s")
    wid = c * NS + s
    zero16 = jnp.zeros((16,), jnp.float32)

    def _zrow(r, _):
        for j in range(F // 16):
            zbuf[r, pl.ds(j * 16, 16)] = zero16
        return 0

    lax.fori_loop(0, ZCH, _zrow, 0)
    row0 = s * ROWS_T
    for k in range(NZ):
        pltpu.sync_copy(zbuf, acc.at[pl.ds(row0 + k * ZCH, ZCH)])
    plsc.subcore_barrier()

    pltpu.sync_copy(src_hbm.at[pl.ds(wid * RW, RW)], idxs)
    pltpu.sync_copy(dst_hbm.at[pl.ds(wid * RW, RW)], idxd)

    def _edge_chunk(j, _):
        pltpu.async_copy(u_hbm.at[idxs.at[j]], rows, sem).wait()
        pltpu.sync_copy(rows, acc.at[idxd.at[j]], add=True)
        return 0

    lax.fori_loop(0, RW, _edge_chunk, 0)
    plsc.subcore_barrier()

    for k in range(NZ):
        r0 = row0 + k * ZCH
        pltpu.sync_copy(acc.at[pl.ds(r0, ZCH)], zbuf)
        pltpu.sync_copy(zbuf, out_hbm.at[c, pl.ds(r0, ZCH)])


@functools.partial(
    pl.kernel,
    out_type=jax.ShapeDtypeStruct((NC, NPAD, F), jnp.float32),
    mesh=_mesh,
    scratch_types=[
        pltpu.VMEM_SHARED((NPAD, F), jnp.float32),
        pltpu.VMEM((RW, C), jnp.int32),
        pltpu.VMEM((C, F), jnp.float32),
        pltpu.VMEM((ZCH, F), jnp.float32),
        pltpu.SemaphoreType.DMA,
    ],
)
def _deg_call(dst_hbm, out_hbm, acc, idxd, ones, zbuf, sem):
    c = lax.axis_index("c")
    s = lax.axis_index("s")
    wid = c * NS + s
    zero16 = jnp.zeros((16,), jnp.float32)
    one16 = jnp.ones((16,), jnp.float32)

    def _zrow(r, _):
        for j in range(F // 16):
            zbuf[r, pl.ds(j * 16, 16)] = zero16
        return 0

    lax.fori_loop(0, ZCH, _zrow, 0)

    def _orow(r, _):
        for j in range(F // 16):
            ones[r, pl.ds(j * 16, 16)] = one16
        return 0

    lax.fori_loop(0, C, _orow, 0)

    row0 = s * ROWS_T
    for k in range(NZ):
        pltpu.sync_copy(zbuf, acc.at[pl.ds(row0 + k * ZCH, ZCH)])
    plsc.subcore_barrier()

    pltpu.sync_copy(dst_hbm.at[pl.ds(wid * RW, RW)], idxd)

    def _edge_chunk(j, _):
        pltpu.sync_copy(ones, acc.at[idxd.at[j]], add=True)
        return 0

    lax.fori_loop(0, RW, _edge_chunk, 0)
    plsc.subcore_barrier()

    for k in range(NZ):
        r0 = row0 + k * ZCH
        pltpu.sync_copy(acc.at[pl.ds(r0, ZCH)], zbuf)
        pltpu.sync_copy(zbuf, out_hbm.at[c, pl.ds(r0, ZCH)])


def _bn_relu(h, gamma, beta):
    mean = jnp.sum(h, axis=0, keepdims=True) * (1.0 / N)
    d = h - mean
    var = jnp.sum(d * d, axis=0, keepdims=True) * (1.0 / N)
    t = gamma * d * lax.rsqrt(var + EPS) + beta
    return jnp.maximum(t, 0.0)


def _k0_body(x_ref, We_ref, be_ref, degp_ref, g_ref, bt_ref, W0_ref,
             h_ref, u_ref, dinv_ref):
    h = jnp.dot(x_ref[...], We_ref[...], preferred_element_type=jnp.float32)
    h = h + be_ref[...]
    dsum = degp_ref[0] + degp_ref[1]
    deg = dsum[:N, 0:1] + 1.0
    dinv = lax.rsqrt(jnp.maximum(deg, 1e-12))
    t = _bn_relu(h, g_ref[...], bt_ref[...])
    u = jnp.dot(t, W0_ref[...], preferred_element_type=jnp.float32) * dinv
    h_ref[...] = h
    u_ref[...] = u
    dinv_ref[...] = dinv


def _kpre_body(h_ref, acc_ref, u_ref, dinv_ref, b_ref, g_ref, bt_ref, W_ref,
               hn_ref, un_ref):
    dinv = dinv_ref[...]
    h = h_ref[...] + dinv * (acc_ref[0][:N] + acc_ref[1][:N] + u_ref[...]) + b_ref[...]
    t = _bn_relu(h, g_ref[...], bt_ref[...])
    un_ref[...] = jnp.dot(t, W_ref[...], preferred_element_type=jnp.float32) * dinv
    hn_ref[...] = h


def _kfin_body(h_ref, acc_ref, u_ref, dinv_ref, b_ref, g_ref, bt_ref,
               lw_ref, lb_ref, o_ref):
    dinv = dinv_ref[...]
    h = h_ref[...] + dinv * (acc_ref[0][:N] + acc_ref[1][:N] + u_ref[...]) + b_ref[...]
    t = _bn_relu(h, g_ref[...], bt_ref[...])
    o_ref[...] = jnp.dot(t, lw_ref[...], preferred_element_type=jnp.float32) + lb_ref[...]


def _f32(*s):
    return jax.ShapeDtypeStruct(s, jnp.float32)


_k0_call = pl.pallas_call(
    _k0_body, out_shape=(_f32(N, F), _f32(N, F), _f32(N, 1)))
_kpre_call = pl.pallas_call(
    _kpre_body, out_shape=(_f32(N, F), _f32(N, F)))
_kfin_call = pl.pallas_call(
    _kfin_body, out_shape=_f32(N, F))


def kernel(x, edge_index, W_enc, b_enc, conv_W, conv_b, bn_gamma, bn_beta,
           lin_W, lin_b):
    src2d = edge_index[0].reshape(IDXROWS, C)
    dst2d = edge_index[1].reshape(IDXROWS, C)
    deg_parts = _deg_call(dst2d)
    h, u, dinv = _k0_call(x, W_enc, b_enc.reshape(1, F), deg_parts,
                          bn_gamma[0].reshape(1, F), bn_beta[0].reshape(1, F),
                          conv_W[0])
    for i in range(L):
        accs = _agg_call(u, src2d, dst2d)
        if i < L - 1:
            h, u = _kpre_call(h, accs, u, dinv, conv_b[i].reshape(1, F),
                              bn_gamma[i + 1].reshape(1, F),
                              bn_beta[i + 1].reshape(1, F), conv_W[i + 1])
        else:
            out = _kfin_call(h, accs, u, dinv, conv_b[i].reshape(1, F),
                             bn_gamma[L - 1].reshape(1, F),
                             bn_beta[L - 1].reshape(1, F), lin_W, lin_b)
    return out


# speedup vs baseline: 22.9128x; 1.3813x over previous
import functools

import jax
import jax.numpy as jnp
from jax import lax
from jax.experimental import pallas as pl
from jax.experimental.pallas import tpu as pltpu
from jax.experimental.pallas import tpu_sc as plsc

N = 10000
E = 320000
F = 128
L = 4
EPS = 1e-5

NC = 2
NS = 16
NW = NC * NS
C = 125
IDXROWS = E // C
RW = IDXROWS // NW
NPAD = 10240
ROWS_T = NPAD // NS
ZCH = 64
NZ = ROWS_T // ZCH
GR = 8
NG = RW // GR

_mesh = plsc.VectorSubcoreMesh(core_axis_name="c", subcore_axis_name="s")


@functools.partial(
    pl.kernel,
    out_type=jax.ShapeDtypeStruct((NC, NPAD, F), jnp.float32),
    mesh=_mesh,
    scratch_types=[
        pltpu.VMEM_SHARED((NPAD, F), jnp.float32),
        pltpu.VMEM((2, GR, C), jnp.int32),
        pltpu.VMEM((2, GR, C), jnp.int32),
        pltpu.VMEM((2, C, F), jnp.float32),
        pltpu.VMEM((ZCH, F), jnp.float32),
        pltpu.SemaphoreType.DMA,
    ],
)
def _agg_call(u_hbm, src_hbm, dst_hbm, out_hbm, acc, idxs, idxd, rows, zbuf, sem):
    c = lax.axis_index("c")
    s = lax.axis_index("s")
    wid = c * NS + s
    zero16 = jnp.zeros((16,), jnp.float32)

    def _zrow(r, _):
        for j in range(F // 16):
            zbuf[r, pl.ds(j * 16, 16)] = zero16
        return 0

    lax.fori_loop(0, ZCH, _zrow, 0)
    row0 = s * ROWS_T
    for k in range(NZ):
        pltpu.sync_copy(zbuf, acc.at[pl.ds(row0 + k * ZCH, ZCH)])
    plsc.subcore_barrier()

    ebase = wid * RW

    def _load_group(g, slot):
        pltpu.sync_copy(src_hbm.at[pl.ds(ebase + g * GR, GR)], idxs.at[slot])
        pltpu.sync_copy(dst_hbm.at[pl.ds(ebase + g * GR, GR)], idxd.at[slot])

    _load_group(0, 0)
    pltpu.async_copy(u_hbm.at[idxs.at[0, 0]], rows.at[0], sem)

    def _group(g, _):
        gs = lax.rem(g, 2)
        ns = lax.rem(g + 1, 2)

        @pl.when(g < NG - 1)
        def _():
            _load_group(g + 1, ns)

        for k in range(GR):
            cur = k % 2
            nxt = (k + 1) % 2
            kn = k + 1
            if kn < GR:
                pltpu.async_copy(u_hbm.at[idxs.at[gs, kn]], rows.at[nxt], sem)
            else:
                pltpu.async_copy(u_hbm.at[idxs.at[ns, 0]], rows.at[nxt], sem)
            pltpu.make_async_copy(u_hbm.at[idxs.at[gs, k]], rows.at[cur], sem).wait()
            pltpu.sync_copy(rows.at[cur], acc.at[idxd.at[gs, k]], add=True)
        return 0

    lax.fori_loop(0, NG, _group, 0)
    pltpu.make_async_copy(u_hbm.at[idxs.at[0, 0]], rows.at[0], sem).wait()
    plsc.subcore_barrier()

    for k in range(NZ):
        r0 = row0 + k * ZCH
        pltpu.sync_copy(acc.at[pl.ds(r0, ZCH)], zbuf)
        pltpu.sync_copy(zbuf, out_hbm.at[c, pl.ds(r0, ZCH)])


@functools.partial(
    pl.kernel,
    out_type=jax.ShapeDtypeStruct((NC, NPAD, F), jnp.float32),
    mesh=_mesh,
    scratch_types=[
        pltpu.VMEM_SHARED((NPAD, F), jnp.float32),
        pltpu.VMEM((RW, C), jnp.int32),
        pltpu.VMEM((C, F), jnp.float32),
        pltpu.VMEM((ZCH, F), jnp.float32),
        pltpu.SemaphoreType.DMA,
    ],
)
def _deg_call(dst_hbm, out_hbm, acc, idxd, ones, zbuf, sem):
    c = lax.axis_index("c")
    s = lax.axis_index("s")
    wid = c * NS + s
    zero16 = jnp.zeros((16,), jnp.float32)
    one16 = jnp.ones((16,), jnp.float32)

    def _zrow(r, _):
        for j in range(F // 16):
            zbuf[r, pl.ds(j * 16, 16)] = zero16
        return 0

    lax.fori_loop(0, ZCH, _zrow, 0)

    def _orow(r, _):
        for j in range(F // 16):
            ones[r, pl.ds(j * 16, 16)] = one16
        return 0

    lax.fori_loop(0, C, _orow, 0)

    row0 = s * ROWS_T
    for k in range(NZ):
        pltpu.sync_copy(zbuf, acc.at[pl.ds(row0 + k * ZCH, ZCH)])
    plsc.subcore_barrier()

    pltpu.sync_copy(dst_hbm.at[pl.ds(wid * RW, RW)], idxd)

    def _edge_chunk(j, _):
        pltpu.sync_copy(ones, acc.at[idxd.at[j]], add=True)
        return 0

    lax.fori_loop(0, RW, _edge_chunk, 0)
    plsc.subcore_barrier()

    for k in range(NZ):
        r0 = row0 + k * ZCH
        pltpu.sync_copy(acc.at[pl.ds(r0, ZCH)], zbuf)
        pltpu.sync_copy(zbuf, out_hbm.at[c, pl.ds(r0, ZCH)])


def _bn_relu(h, gamma, beta):
    mean = jnp.sum(h, axis=0, keepdims=True) * (1.0 / N)
    d = h - mean
    var = jnp.sum(d * d, axis=0, keepdims=True) * (1.0 / N)
    t = gamma * d * lax.rsqrt(var + EPS) + beta
    return jnp.maximum(t, 0.0)


def _k0_body(x_ref, We_ref, be_ref, degp_ref, g_ref, bt_ref, W0_ref,
             h_ref, u_ref, dinv_ref):
    h = jnp.dot(x_ref[...], We_ref[...], preferred_element_type=jnp.float32)
    h = h + be_ref[...]
    dsum = degp_ref[0] + degp_ref[1]
    deg = dsum[:N, 0:1] + 1.0
    dinv = lax.rsqrt(jnp.maximum(deg, 1e-12))
    t = _bn_relu(h, g_ref[...], bt_ref[...])
    u = jnp.dot(t, W0_ref[...], preferred_element_type=jnp.float32) * dinv
    h_ref[...] = h
    u_ref[...] = u
    dinv_ref[...] = dinv


def _kpre_body(h_ref, acc_ref, u_ref, dinv_ref, b_ref, g_ref, bt_ref, W_ref,
               hn_ref, un_ref):
    dinv = dinv_ref[...]
    h = h_ref[...] + dinv * (acc_ref[0][:N] + acc_ref[1][:N] + u_ref[...]) + b_ref[...]
    t = _bn_relu(h, g_ref[...], bt_ref[...])
    un_ref[...] = jnp.dot(t, W_ref[...], preferred_element_type=jnp.float32) * dinv
    hn_ref[...] = h


def _kfin_body(h_ref, acc_ref, u_ref, dinv_ref, b_ref, g_ref, bt_ref,
               lw_ref, lb_ref, o_ref):
    dinv = dinv_ref[...]
    h = h_ref[...] + dinv * (acc_ref[0][:N] + acc_ref[1][:N] + u_ref[...]) + b_ref[...]
    t = _bn_relu(h, g_ref[...], bt_ref[...])
    o_ref[...] = jnp.dot(t, lw_ref[...], preferred_element_type=jnp.float32) + lb_ref[...]


def _f32(*s):
    return jax.ShapeDtypeStruct(s, jnp.float32)


_k0_call = pl.pallas_call(
    _k0_body, out_shape=(_f32(N, F), _f32(N, F), _f32(N, 1)))
_kpre_call = pl.pallas_call(
    _kpre_body, out_shape=(_f32(N, F), _f32(N, F)))
_kfin_call = pl.pallas_call(
    _kfin_body, out_shape=_f32(N, F))


def kernel(x, edge_index, W_enc, b_enc, conv_W, conv_b, bn_gamma, bn_beta,
           lin_W, lin_b):
    src2d = edge_index[0].reshape(IDXROWS, C)
    dst2d = edge_index[1].reshape(IDXROWS, C)
    deg_parts = _deg_call(dst2d)
    h, u, dinv = _k0_call(x, W_enc, b_enc.reshape(1, F), deg_parts,
                          bn_gamma[0].reshape(1, F), bn_beta[0].reshape(1, F),
                          conv_W[0])
    for i in range(L):
        accs = _agg_call(u, src2d, dst2d)
        if i < L - 1:
            h, u = _kpre_call(h, accs, u, dinv, conv_b[i].reshape(1, F),
                              bn_gamma[i + 1].reshape(1, F),
                              bn_beta[i + 1].reshape(1, F), conv_W[i + 1])
        else:
            out = _kfin_call(h, accs, u, dinv, conv_b[i].reshape(1, F),
                             bn_gamma[L - 1].reshape(1, F),
                             bn_beta[L - 1].reshape(1, F), lin_W, lin_b)
    return out


# speedup vs baseline: 24.4679x; 1.0679x over previous
import functools

import jax
import jax.numpy as jnp
from jax import lax
from jax.experimental import pallas as pl
from jax.experimental.pallas import tpu as pltpu
from jax.experimental.pallas import tpu_sc as plsc

N = 10000
E = 320000
F = 128
L = 4
EPS = 1e-5

NC = 2
NS = 16
NW = NC * NS
C = 125
IDXROWS = E // C
RW = IDXROWS // NW
NPAD = 10240
ROWS_T = NPAD // NS
ZCH = 64
NZ = ROWS_T // ZCH
GR = 8
NG = RW // GR

_mesh = plsc.VectorSubcoreMesh(core_axis_name="c", subcore_axis_name="s")


@functools.partial(
    pl.kernel,
    out_type=jax.ShapeDtypeStruct((NC, NPAD, F), jnp.float32),
    mesh=_mesh,
    scratch_types=[
        pltpu.VMEM_SHARED((NPAD, F), jnp.float32),
        pltpu.VMEM((2, GR, C), jnp.int32),
        pltpu.VMEM((2, GR, C), jnp.int32),
        pltpu.VMEM((2, C, F), jnp.float32),
        pltpu.VMEM((ZCH, F), jnp.float32),
        pltpu.SemaphoreType.DMA,
        pltpu.SemaphoreType.DMA,
    ],
)
def _agg_call(u_hbm, src_hbm, dst_hbm, out_hbm, acc, idxs, idxd, rows, zbuf,
              sem, isem):
    c = lax.axis_index("c")
    s = lax.axis_index("s")
    wid = c * NS + s
    ebase = wid * RW
    zero16 = jnp.zeros((16,), jnp.float32)

    def _start_group(g, slot):
        pltpu.async_copy(src_hbm.at[pl.ds(ebase + g * GR, GR)], idxs.at[slot], isem)
        pltpu.async_copy(dst_hbm.at[pl.ds(ebase + g * GR, GR)], idxd.at[slot], isem)

    def _wait_group(g, slot):
        pltpu.make_async_copy(src_hbm.at[pl.ds(ebase + g * GR, GR)], idxs.at[slot], isem).wait()
        pltpu.make_async_copy(dst_hbm.at[pl.ds(ebase + g * GR, GR)], idxd.at[slot], isem).wait()

    _start_group(0, 0)

    def _zrow(r, _):
        for j in range(F // 16):
            zbuf[r, pl.ds(j * 16, 16)] = zero16
        return 0

    lax.fori_loop(0, ZCH, _zrow, 0)
    row0 = s * ROWS_T
    for k in range(NZ):
        pltpu.sync_copy(zbuf, acc.at[pl.ds(row0 + k * ZCH, ZCH)])

    _wait_group(0, 0)
    pltpu.async_copy(u_hbm.at[idxs.at[0, 0]], rows.at[0], sem)
    plsc.subcore_barrier()

    def _group(g, _):
        gs = lax.rem(g, 2)
        ns = lax.rem(g + 1, 2)

        @pl.when(g < NG - 1)
        def _():
            _start_group(g + 1, ns)

        for k in range(GR):
            cur = k % 2
            nxt = (k + 1) % 2
            kn = k + 1
            if kn < GR:
                pltpu.async_copy(u_hbm.at[idxs.at[gs, kn]], rows.at[nxt], sem)
            else:
                @pl.when(g < NG - 1)
                def _():
                    _wait_group(g + 1, ns)

                pltpu.async_copy(u_hbm.at[idxs.at[ns, 0]], rows.at[nxt], sem)
            pltpu.make_async_copy(u_hbm.at[idxs.at[gs, k]], rows.at[cur], sem).wait()
            pltpu.sync_copy(rows.at[cur], acc.at[idxd.at[gs, k]], add=True)
        return 0

    lax.fori_loop(0, NG, _group, 0)
    pltpu.make_async_copy(u_hbm.at[idxs.at[0, 0]], rows.at[0], sem).wait()
    plsc.subcore_barrier()

    for k in range(NZ):
        r0 = row0 + k * ZCH
        pltpu.sync_copy(acc.at[pl.ds(r0, ZCH)], zbuf)
        pltpu.sync_copy(zbuf, out_hbm.at[c, pl.ds(r0, ZCH)])


@functools.partial(
    pl.kernel,
    out_type=jax.ShapeDtypeStruct((NC, NPAD, F), jnp.float32),
    mesh=_mesh,
    scratch_types=[
        pltpu.VMEM_SHARED((NPAD, F), jnp.float32),
        pltpu.VMEM((RW, C), jnp.int32),
        pltpu.VMEM((C, F), jnp.float32),
        pltpu.VMEM((ZCH, F), jnp.float32),
        pltpu.SemaphoreType.DMA,
    ],
)
def _deg_call(dst_hbm, out_hbm, acc, idxd, ones, zbuf, sem):
    c = lax.axis_index("c")
    s = lax.axis_index("s")
    wid = c * NS + s
    zero16 = jnp.zeros((16,), jnp.float32)
    one16 = jnp.ones((16,), jnp.float32)

    def _zrow(r, _):
        for j in range(F // 16):
            zbuf[r, pl.ds(j * 16, 16)] = zero16
        return 0

    lax.fori_loop(0, ZCH, _zrow, 0)

    def _orow(r, _):
        for j in range(F // 16):
            ones[r, pl.ds(j * 16, 16)] = one16
        return 0

    lax.fori_loop(0, C, _orow, 0)

    row0 = s * ROWS_T
    for k in range(NZ):
        pltpu.sync_copy(zbuf, acc.at[pl.ds(row0 + k * ZCH, ZCH)])
    plsc.subcore_barrier()

    pltpu.sync_copy(dst_hbm.at[pl.ds(wid * RW, RW)], idxd)

    def _edge_chunk(j, _):
        pltpu.sync_copy(ones, acc.at[idxd.at[j]], add=True)
        return 0

    lax.fori_loop(0, RW, _edge_chunk, 0)
    plsc.subcore_barrier()

    for k in range(NZ):
        r0 = row0 + k * ZCH
        pltpu.sync_copy(acc.at[pl.ds(r0, ZCH)], zbuf)
        pltpu.sync_copy(zbuf, out_hbm.at[c, pl.ds(r0, ZCH)])


def _bn_relu(h, gamma, beta):
    mean = jnp.sum(h, axis=0, keepdims=True) * (1.0 / N)
    d = h - mean
    var = jnp.sum(d * d, axis=0, keepdims=True) * (1.0 / N)
    t = gamma * d * lax.rsqrt(var + EPS) + beta
    return jnp.maximum(t, 0.0)


def _k0_body(x_ref, We_ref, be_ref, degp_ref, g_ref, bt_ref, W0_ref,
             h_ref, u_ref, dinv_ref):
    h = jnp.dot(x_ref[...], We_ref[...], preferred_element_type=jnp.float32)
    h = h + be_ref[...]
    dsum = degp_ref[0] + degp_ref[1]
    deg = dsum[:N, 0:1] + 1.0
    dinv = lax.rsqrt(jnp.maximum(deg, 1e-12))
    t = _bn_relu(h, g_ref[...], bt_ref[...])
    u = jnp.dot(t, W0_ref[...], preferred_element_type=jnp.float32) * dinv
    h_ref[...] = h
    u_ref[...] = u
    dinv_ref[...] = dinv


def _kpre_body(h_ref, acc_ref, u_ref, dinv_ref, b_ref, g_ref, bt_ref, W_ref,
               hn_ref, un_ref):
    dinv = dinv_ref[...]
    h = h_ref[...] + dinv * (acc_ref[0][:N] + acc_ref[1][:N] + u_ref[...]) + b_ref[...]
    t = _bn_relu(h, g_ref[...], bt_ref[...])
    un_ref[...] = jnp.dot(t, W_ref[...], preferred_element_type=jnp.float32) * dinv
    hn_ref[...] = h


def _kfin_body(h_ref, acc_ref, u_ref, dinv_ref, b_ref, g_ref, bt_ref,
               lw_ref, lb_ref, o_ref):
    dinv = dinv_ref[...]
    h = h_ref[...] + dinv * (acc_ref[0][:N] + acc_ref[1][:N] + u_ref[...]) + b_ref[...]
    t = _bn_relu(h, g_ref[...], bt_ref[...])
    o_ref[...] = jnp.dot(t, lw_ref[...], preferred_element_type=jnp.float32) + lb_ref[...]


def _f32(*s):
    return jax.ShapeDtypeStruct(s, jnp.float32)


_k0_call = pl.pallas_call(
    _k0_body, out_shape=(_f32(N, F), _f32(N, F), _f32(N, 1)))
_kpre_call = pl.pallas_call(
    _kpre_body, out_shape=(_f32(N, F), _f32(N, F)))
_kfin_call = pl.pallas_call(
    _kfin_body, out_shape=_f32(N, F))


def kernel(x, edge_index, W_enc, b_enc, conv_W, conv_b, bn_gamma, bn_beta,
           lin_W, lin_b):
    src2d = edge_index[0].reshape(IDXROWS, C)
    dst2d = edge_index[1].reshape(IDXROWS, C)
    deg_parts = _deg_call(dst2d)
    h, u, dinv = _k0_call(x, W_enc, b_enc.reshape(1, F), deg_parts,
                          bn_gamma[0].reshape(1, F), bn_beta[0].reshape(1, F),
                          conv_W[0])
    for i in range(L):
        accs = _agg_call(u, src2d, dst2d)
        if i < L - 1:
            h, u = _kpre_call(h, accs, u, dinv, conv_b[i].reshape(1, F),
                              bn_gamma[i + 1].reshape(1, F),
                              bn_beta[i + 1].reshape(1, F), conv_W[i + 1])
        else:
            out = _kfin_call(h, accs, u, dinv, conv_b[i].reshape(1, F),
                             bn_gamma[L - 1].reshape(1, F),
                             bn_beta[L - 1].reshape(1, F), lin_W, lin_b)
    return out


# speedup vs baseline: 24.7252x; 1.0105x over previous
import functools

import jax
import jax.numpy as jnp
from jax import lax
from jax.experimental import pallas as pl
from jax.experimental.pallas import tpu as pltpu
from jax.experimental.pallas import tpu_sc as plsc

N = 10000
E = 320000
F = 128
L = 4
EPS = 1e-5

NC = 2
NS = 16
NW = NC * NS
C = 125
IDXROWS = E // C
RW = IDXROWS // NW
NPAD = 10240
ROWS_T = NPAD // NS
ZCH = 64
NZ = ROWS_T // ZCH
GR = 8
NG = RW // GR

_mesh = plsc.VectorSubcoreMesh(core_axis_name="c", subcore_axis_name="s")


@functools.partial(
    pl.kernel,
    out_type=jax.ShapeDtypeStruct((NC, NPAD, F), jnp.float32),
    mesh=_mesh,
    scratch_types=[
        pltpu.VMEM_SHARED((NPAD, F), jnp.float32),
        pltpu.VMEM((2, GR, C), jnp.int32),
        pltpu.VMEM((2, GR, C), jnp.int32),
        pltpu.VMEM((2, C, F), jnp.float32),
        pltpu.VMEM((ZCH, F), jnp.float32),
        pltpu.SemaphoreType.DMA,
        pltpu.SemaphoreType.DMA,
        pltpu.SemaphoreType.DMA,
    ],
)
def _agg_call(u_hbm, src_hbm, dst_hbm, out_hbm, acc, idxs, idxd, rows, zbuf,
              sem, isem, ssem):
    c = lax.axis_index("c")
    s = lax.axis_index("s")
    wid = c * NS + s
    ebase = wid * RW
    zero16 = jnp.zeros((16,), jnp.float32)

    def _start_group(g, slot):
        pltpu.async_copy(src_hbm.at[pl.ds(ebase + g * GR, GR)], idxs.at[slot], isem)
        pltpu.async_copy(dst_hbm.at[pl.ds(ebase + g * GR, GR)], idxd.at[slot], isem)

    def _wait_group(g, slot):
        pltpu.make_async_copy(src_hbm.at[pl.ds(ebase + g * GR, GR)], idxs.at[slot], isem).wait()
        pltpu.make_async_copy(dst_hbm.at[pl.ds(ebase + g * GR, GR)], idxd.at[slot], isem).wait()

    _start_group(0, 0)

    def _zrow(r, _):
        for j in range(F // 16):
            zbuf[r, pl.ds(j * 16, 16)] = zero16
        return 0

    lax.fori_loop(0, ZCH, _zrow, 0)
    row0 = s * ROWS_T
    for k in range(NZ):
        pltpu.sync_copy(zbuf, acc.at[pl.ds(row0 + k * ZCH, ZCH)])

    _wait_group(0, 0)
    pltpu.async_copy(u_hbm.at[idxs.at[0, 0]], rows.at[0], sem)
    plsc.subcore_barrier()

    def _group(g, _):
        gs = lax.rem(g, 2)
        ns = lax.rem(g + 1, 2)

        @pl.when(g < NG - 1)
        def _():
            _start_group(g + 1, ns)

        for k in range(GR):
            cur = k % 2
            nxt = (k + 1) % 2
            if k == 0:
                @pl.when(g > 0)
                def _():
                    pltpu.make_async_copy(
                        rows.at[nxt], acc.at[idxd.at[gs, 0]], ssem).wait()
            else:
                pltpu.make_async_copy(
                    rows.at[nxt], acc.at[idxd.at[gs, k]], ssem).wait()
            kn = k + 1
            if kn < GR:
                pltpu.async_copy(u_hbm.at[idxs.at[gs, kn]], rows.at[nxt], sem)
            else:
                @pl.when(g < NG - 1)
                def _():
                    _wait_group(g + 1, ns)

                pltpu.async_copy(u_hbm.at[idxs.at[ns, 0]], rows.at[nxt], sem)
            pltpu.make_async_copy(u_hbm.at[idxs.at[gs, k]], rows.at[cur], sem).wait()
            pltpu.async_copy(rows.at[cur], acc.at[idxd.at[gs, k]], ssem, add=True)
        return 0

    lax.fori_loop(0, NG, _group, 0)
    pltpu.make_async_copy(u_hbm.at[idxs.at[0, 0]], rows.at[0], sem).wait()
    pltpu.make_async_copy(rows.at[1], acc.at[idxd.at[0, 0]], ssem).wait()
    plsc.subcore_barrier()

    pltpu.sync_copy(acc.at[pl.ds(row0, ROWS_T)], out_hbm.at[c, pl.ds(row0, ROWS_T)])


@functools.partial(
    pl.kernel,
    out_type=jax.ShapeDtypeStruct((NC, NPAD, F), jnp.float32),
    mesh=_mesh,
    scratch_types=[
        pltpu.VMEM_SHARED((NPAD, F), jnp.float32),
        pltpu.VMEM((RW, C), jnp.int32),
        pltpu.VMEM((C, F), jnp.float32),
        pltpu.VMEM((ZCH, F), jnp.float32),
        pltpu.SemaphoreType.DMA,
    ],
)
def _deg_call(dst_hbm, out_hbm, acc, idxd, ones, zbuf, sem):
    c = lax.axis_index("c")
    s = lax.axis_index("s")
    wid = c * NS + s
    zero16 = jnp.zeros((16,), jnp.float32)
    one16 = jnp.ones((16,), jnp.float32)

    def _zrow(r, _):
        for j in range(F // 16):
            zbuf[r, pl.ds(j * 16, 16)] = zero16
        return 0

    lax.fori_loop(0, ZCH, _zrow, 0)

    def _orow(r, _):
        for j in range(F // 16):
            ones[r, pl.ds(j * 16, 16)] = one16
        return 0

    lax.fori_loop(0, C, _orow, 0)

    row0 = s * ROWS_T
    for k in range(NZ):
        pltpu.sync_copy(zbuf, acc.at[pl.ds(row0 + k * ZCH, ZCH)])
    plsc.subcore_barrier()

    pltpu.sync_copy(dst_hbm.at[pl.ds(wid * RW, RW)], idxd)

    def _edge_chunk(j, _):
        pltpu.sync_copy(ones, acc.at[idxd.at[j]], add=True)
        return 0

    lax.fori_loop(0, RW, _edge_chunk, 0)
    plsc.subcore_barrier()

    pltpu.sync_copy(acc.at[pl.ds(row0, ROWS_T)], out_hbm.at[c, pl.ds(row0, ROWS_T)])


def _bn_relu(h, gamma, beta):
    mean = jnp.sum(h, axis=0, keepdims=True) * (1.0 / N)
    d = h - mean
    var = jnp.sum(d * d, axis=0, keepdims=True) * (1.0 / N)
    t = gamma * d * lax.rsqrt(var + EPS) + beta
    return jnp.maximum(t, 0.0)


def _k0_body(x_ref, We_ref, be_ref, degp_ref, g_ref, bt_ref, W0_ref,
             h_ref, u_ref, dinv_ref):
    h = jnp.dot(x_ref[...], We_ref[...], preferred_element_type=jnp.float32)
    h = h + be_ref[...]
    dsum = degp_ref[0] + degp_ref[1]
    deg = dsum[:N, 0:1] + 1.0
    dinv = lax.rsqrt(jnp.maximum(deg, 1e-12))
    t = _bn_relu(h, g_ref[...], bt_ref[...])
    u = jnp.dot(t, W0_ref[...], preferred_element_type=jnp.float32) * dinv
    h_ref[...] = h
    u_ref[...] = u
    dinv_ref[...] = dinv


def _kpre_body(h_ref, acc_ref, u_ref, dinv_ref, b_ref, g_ref, bt_ref, W_ref,
               hn_ref, un_ref):
    dinv = dinv_ref[...]
    h = h_ref[...] + dinv * (acc_ref[0][:N] + acc_ref[1][:N] + u_ref[...]) + b_ref[...]
    t = _bn_relu(h, g_ref[...], bt_ref[...])
    un_ref[...] = jnp.dot(t, W_ref[...], preferred_element_type=jnp.float32) * dinv
    hn_ref[...] = h


def _kfin_body(h_ref, acc_ref, u_ref, dinv_ref, b_ref, g_ref, bt_ref,
               lw_ref, lb_ref, o_ref):
    dinv = dinv_ref[...]
    h = h_ref[...] + dinv * (acc_ref[0][:N] + acc_ref[1][:N] + u_ref[...]) + b_ref[...]
    t = _bn_relu(h, g_ref[...], bt_ref[...])
    o_ref[...] = jnp.dot(t, lw_ref[...], preferred_element_type=jnp.float32) + lb_ref[...]


def _f32(*s):
    return jax.ShapeDtypeStruct(s, jnp.float32)


_k0_call = pl.pallas_call(
    _k0_body, out_shape=(_f32(N, F), _f32(N, F), _f32(N, 1)))
_kpre_call = pl.pallas_call(
    _kpre_body, out_shape=(_f32(N, F), _f32(N, F)))
_kfin_call = pl.pallas_call(
    _kfin_body, out_shape=_f32(N, F))


def kernel(x, edge_index, W_enc, b_enc, conv_W, conv_b, bn_gamma, bn_beta,
           lin_W, lin_b):
    src2d = edge_index[0].reshape(IDXROWS, C)
    dst2d = edge_index[1].reshape(IDXROWS, C)
    deg_parts = _deg_call(dst2d)
    h, u, dinv = _k0_call(x, W_enc, b_enc.reshape(1, F), deg_parts,
                          bn_gamma[0].reshape(1, F), bn_beta[0].reshape(1, F),
                          conv_W[0])
    for i in range(L):
        accs = _agg_call(u, src2d, dst2d)
        if i < L - 1:
            h, u = _kpre_call(h, accs, u, dinv, conv_b[i].reshape(1, F),
                              bn_gamma[i + 1].reshape(1, F),
                              bn_beta[i + 1].reshape(1, F), conv_W[i + 1])
        else:
            out = _kfin_call(h, accs, u, dinv, conv_b[i].reshape(1, F),
                             bn_gamma[L - 1].reshape(1, F),
                             bn_beta[L - 1].reshape(1, F), lin_W, lin_b)
    return out


# speedup vs baseline: 27.2381x; 1.1016x over previous
import functools

import jax
import jax.numpy as jnp
from jax import lax
from jax.experimental import pallas as pl
from jax.experimental.pallas import tpu as pltpu
from jax.experimental.pallas import tpu_sc as plsc

N = 10000
E = 320000
F = 128
L = 4
EPS = 1e-5

NC = 2
NS = 16
NW = NC * NS
C = 125
IDXROWS = E // C
RW = IDXROWS // NW
NPAD = 10240
ROWS_T = NPAD // NS
ZCH = 64
NZ = ROWS_T // ZCH
GR = 8
NG = RW // GR

_mesh = plsc.VectorSubcoreMesh(core_axis_name="c", subcore_axis_name="s")


@functools.partial(
    pl.kernel,
    out_type=jax.ShapeDtypeStruct((NC, NPAD, F), jnp.float32),
    mesh=_mesh,
    scratch_types=[
        pltpu.VMEM_SHARED((NPAD, F), jnp.float32),
        pltpu.VMEM((2, GR, C), jnp.int32),
        pltpu.VMEM((2, GR, C), jnp.int32),
        pltpu.VMEM((2, C, F), jnp.float32),
        pltpu.VMEM((ZCH, F), jnp.float32),
        pltpu.SemaphoreType.DMA,
        pltpu.SemaphoreType.DMA,
        pltpu.SemaphoreType.DMA,
    ],
)
def _agg_call(u_hbm, src_hbm, dst_hbm, out_hbm, acc, idxs, idxd, rows, zbuf,
              sem, isem, ssem):
    c = lax.axis_index("c")
    s = lax.axis_index("s")
    wid = c * NS + s
    ebase = wid * RW
    zero16 = jnp.zeros((16,), jnp.float32)

    def _start_group(g, slot):
        pltpu.async_copy(src_hbm.at[pl.ds(ebase + g * GR, GR)], idxs.at[slot], isem)
        pltpu.async_copy(dst_hbm.at[pl.ds(ebase + g * GR, GR)], idxd.at[slot], isem)

    def _wait_group(g, slot):
        pltpu.make_async_copy(src_hbm.at[pl.ds(ebase + g * GR, GR)], idxs.at[slot], isem).wait()
        pltpu.make_async_copy(dst_hbm.at[pl.ds(ebase + g * GR, GR)], idxd.at[slot], isem).wait()

    _start_group(0, 0)

    def _zrow(r, _):
        for j in range(F // 16):
            zbuf[r, pl.ds(j * 16, 16)] = zero16
        return 0

    lax.fori_loop(0, ZCH, _zrow, 0)
    row0 = s * ROWS_T
    for k in range(NZ):
        pltpu.sync_copy(zbuf, acc.at[pl.ds(row0 + k * ZCH, ZCH)])

    _wait_group(0, 0)
    pltpu.async_copy(u_hbm.at[idxs.at[0, 0]], rows.at[0], sem)
    plsc.subcore_barrier()

    def _group(g, _):
        gs = lax.rem(g, 2)
        ns = lax.rem(g + 1, 2)

        @pl.when(g < NG - 1)
        def _():
            _start_group(g + 1, ns)

        for k in range(GR):
            cur = k % 2
            nxt = (k + 1) % 2
            if k == 0:
                @pl.when(g > 0)
                def _():
                    pltpu.make_async_copy(
                        rows.at[nxt], acc.at[idxd.at[gs, 0]], ssem).wait()
            else:
                pltpu.make_async_copy(
                    rows.at[nxt], acc.at[idxd.at[gs, k]], ssem).wait()
            kn = k + 1
            if kn < GR:
                pltpu.async_copy(u_hbm.at[idxs.at[gs, kn]], rows.at[nxt], sem)
            else:
                @pl.when(g < NG - 1)
                def _():
                    _wait_group(g + 1, ns)

                pltpu.async_copy(u_hbm.at[idxs.at[ns, 0]], rows.at[nxt], sem)
            pltpu.make_async_copy(u_hbm.at[idxs.at[gs, k]], rows.at[cur], sem).wait()
            pltpu.async_copy(rows.at[cur], acc.at[idxd.at[gs, k]], ssem, add=True)
        return 0

    lax.fori_loop(0, NG, _group, 0)
    pltpu.make_async_copy(u_hbm.at[idxs.at[0, 0]], rows.at[0], sem).wait()
    pltpu.make_async_copy(rows.at[1], acc.at[idxd.at[0, 0]], ssem).wait()
    plsc.subcore_barrier()

    pltpu.sync_copy(acc.at[pl.ds(row0, ROWS_T)], out_hbm.at[c, pl.ds(row0, ROWS_T)])


EPW = E // NW


@functools.partial(
    pl.kernel,
    out_type=jax.ShapeDtypeStruct((NC, NS, NPAD), jnp.float32),
    mesh=_mesh,
    scratch_types=[
        pltpu.VMEM((EPW,), jnp.int32),
        pltpu.VMEM((NPAD,), jnp.float32),
    ],
    compiler_params=pltpu.CompilerParams(needs_layout_passes=False),
)
def _deg_call(dst_hbm, out_hbm, idxd, hist):
    c = lax.axis_index("c")
    s = lax.axis_index("s")
    wid = c * NS + s
    zero16 = jnp.zeros((16,), jnp.float32)
    one16 = jnp.ones((16,), jnp.float32)

    pltpu.sync_copy(dst_hbm.at[pl.ds(wid * EPW, EPW)], idxd)

    def _zchunk(i, _):
        hist[pl.ds(i * 16, 16)] = zero16
        return 0

    lax.fori_loop(0, NPAD // 16, _zchunk, 0)

    def _echunk(i, _):
        v = idxd[pl.ds(i * 16, 16)]
        plsc.addupdate_scatter(hist, [v], one16)
        return 0

    lax.fori_loop(0, EPW // 16, _echunk, 0)
    pltpu.sync_copy(hist, out_hbm.at[c, s])


def _bn_relu(h, gamma, beta):
    mean = jnp.sum(h, axis=0, keepdims=True) * (1.0 / N)
    d = h - mean
    var = jnp.sum(d * d, axis=0, keepdims=True) * (1.0 / N)
    t = gamma * d * lax.rsqrt(var + EPS) + beta
    return jnp.maximum(t, 0.0)


def _ksum_body(degp_ref, dinv_ref):
    dsum = jnp.sum(degp_ref[...], axis=(0, 1))
    dinv_ref[...] = lax.rsqrt(dsum + 1.0)


def _k0_body(x_ref, We_ref, be_ref, dinvc_ref, g_ref, bt_ref, W0_ref,
             h_ref, u_ref, dinv_ref):
    h = jnp.dot(x_ref[...], We_ref[...], preferred_element_type=jnp.float32)
    h = h + be_ref[...]
    dinv = dinvc_ref[...][:N]
    t = _bn_relu(h, g_ref[...], bt_ref[...])
    u = jnp.dot(t, W0_ref[...], preferred_element_type=jnp.float32) * dinv
    h_ref[...] = h
    u_ref[...] = u
    dinv_ref[...] = dinv


def _kpre_body(h_ref, acc_ref, u_ref, dinv_ref, b_ref, g_ref, bt_ref, W_ref,
               hn_ref, un_ref):
    dinv = dinv_ref[...]
    h = h_ref[...] + dinv * (acc_ref[0][:N] + acc_ref[1][:N] + u_ref[...]) + b_ref[...]
    t = _bn_relu(h, g_ref[...], bt_ref[...])
    un_ref[...] = jnp.dot(t, W_ref[...], preferred_element_type=jnp.float32) * dinv
    hn_ref[...] = h


def _kfin_body(h_ref, acc_ref, u_ref, dinv_ref, b_ref, g_ref, bt_ref,
               lw_ref, lb_ref, o_ref):
    dinv = dinv_ref[...]
    h = h_ref[...] + dinv * (acc_ref[0][:N] + acc_ref[1][:N] + u_ref[...]) + b_ref[...]
    t = _bn_relu(h, g_ref[...], bt_ref[...])
    o_ref[...] = jnp.dot(t, lw_ref[...], preferred_element_type=jnp.float32) + lb_ref[...]


def _f32(*s):
    return jax.ShapeDtypeStruct(s, jnp.float32)


_ksum_call = pl.pallas_call(_ksum_body, out_shape=_f32(NPAD))
_k0_call = pl.pallas_call(
    _k0_body, out_shape=(_f32(N, F), _f32(N, F), _f32(N, 1)))
_kpre_call = pl.pallas_call(
    _kpre_body, out_shape=(_f32(N, F), _f32(N, F)))
_kfin_call = pl.pallas_call(
    _kfin_body, out_shape=_f32(N, F))


def kernel(x, edge_index, W_enc, b_enc, conv_W, conv_b, bn_gamma, bn_beta,
           lin_W, lin_b):
    src2d = edge_index[0].reshape(IDXROWS, C)
    dst2d = edge_index[1].reshape(IDXROWS, C)
    deg_parts = _deg_call(edge_index[1])
    dinv_col = _ksum_call(deg_parts).reshape(NPAD, 1)
    h, u, dinv = _k0_call(x, W_enc, b_enc.reshape(1, F), dinv_col,
                          bn_gamma[0].reshape(1, F), bn_beta[0].reshape(1, F),
                          conv_W[0])
    for i in range(L):
        accs = _agg_call(u, src2d, dst2d)
        if i < L - 1:
            h, u = _kpre_call(h, accs, u, dinv, conv_b[i].reshape(1, F),
                              bn_gamma[i + 1].reshape(1, F),
                              bn_beta[i + 1].reshape(1, F), conv_W[i + 1])
        else:
            out = _kfin_call(h, accs, u, dinv, conv_b[i].reshape(1, F),
                             bn_gamma[L - 1].reshape(1, F),
                             bn_beta[L - 1].reshape(1, F), lin_W, lin_b)
    return out
